# baseline (speedup 1.0000x reference)
import jax
import jax.numpy as jnp
from jax.experimental import pallas as pl
from jax.experimental.pallas import tpu as pltpu
from jax.experimental.pallas import tpu_sc as plsc

_GATHER_WINDOW = 128
_VOCAB_TILE = 512


def _sc_gather(table, indices):
    b = indices.shape[0]
    d = table.shape[1]
    idx2d = indices.reshape(1, b)
    mesh = plsc.VectorSubcoreMesh(core_axis_name="core", subcore_axis_name="subcore")

    @pl.kernel(out_type=jax.ShapeDtypeStruct((b, d), table.dtype), mesh=mesh)
    def gather_kernel(x_hbm, i_hbm, o_hbm):
        def body(i_vmem, o_vmem):
            pltpu.sync_copy(x_hbm.at[i_vmem.at[0]], o_vmem)

        pltpu.emit_pipeline(
            body,
            grid=(b // _GATHER_WINDOW,),
            in_specs=[pl.BlockSpec((1, _GATHER_WINDOW), index_map=lambda i: (0, i))],
            out_specs=[pl.BlockSpec((_GATHER_WINDOW, d), index_map=lambda i: (i, 0))],
            core_axis_name=("core", "subcore"),
            dimension_semantics=(pltpu.PARALLEL,),
        )(i_hbm, o_hbm)

    return gather_kernel(table, idx2d)


def _matmul_body(c_ref, e_ref, o_ref):
    o_ref[...] = jax.lax.dot_general(
        c_ref[...],
        e_ref[...],
        dimension_numbers=(((1,), (1,)), ((), ())),
        preferred_element_type=jnp.float32,
        precision=jax.lax.Precision.DEFAULT,
    )


def kernel(center_words, in_emb, out_emb):
    b = center_words.shape[0]
    v, d = out_emb.shape

    in_pad = jnp.pad(in_emb, ((0, 0), (0, 128 - d)))
    center_vecs = _sc_gather(in_pad, center_words)

    center_bf = center_vecs[:, :d].astype(jnp.bfloat16)
    out_bf = out_emb.astype(jnp.bfloat16)

    grid = (pl.cdiv(v, _VOCAB_TILE),)
    scores = pl.pallas_call(
        _matmul_body,
        grid=grid,
        in_specs=[
            pl.BlockSpec((b, d), lambda j: (0, 0)),
            pl.BlockSpec((_VOCAB_TILE, d), lambda j: (j, 0)),
        ],
        out_specs=pl.BlockSpec((b, _VOCAB_TILE), lambda j: (0, j)),
        out_shape=jax.ShapeDtypeStruct((b, v), jnp.float32),
    )(center_bf, out_bf)
    return scores


# speedup vs baseline: 1.0024x; 1.0024x over previous
import jax
import jax.numpy as jnp
from jax.experimental import pallas as pl
from jax.experimental.pallas import tpu as pltpu
from jax.experimental.pallas import tpu_sc as plsc

_GATHER_WINDOW = 128
_VOCAB_TILE = 1024


def _sc_gather(table, indices):
    b = indices.shape[0]
    d = table.shape[1]
    idx2d = indices.reshape(1, b)
    mesh = plsc.VectorSubcoreMesh(core_axis_name="core", subcore_axis_name="subcore")

    @pl.kernel(out_type=jax.ShapeDtypeStruct((b, d), table.dtype), mesh=mesh)
    def gather_kernel(x_hbm, i_hbm, o_hbm):
        def body(i_vmem, o_vmem):
            pltpu.sync_copy(x_hbm.at[i_vmem.at[0]], o_vmem)

        pltpu.emit_pipeline(
            body,
            grid=(b // _GATHER_WINDOW,),
            in_specs=[pl.BlockSpec((1, _GATHER_WINDOW), index_map=lambda i: (0, i))],
            out_specs=[pl.BlockSpec((_GATHER_WINDOW, d), index_map=lambda i: (i, 0))],
            core_axis_name=("core", "subcore"),
            dimension_semantics=(pltpu.PARALLEL,),
        )(i_hbm, o_hbm)

    return gather_kernel(table, idx2d)


def _matmul_body(c_ref, e_ref, o_ref):
    o_ref[...] = jax.lax.dot_general(
        c_ref[...],
        e_ref[...],
        dimension_numbers=(((1,), (1,)), ((), ())),
        preferred_element_type=jnp.float32,
        precision=jax.lax.Precision.DEFAULT,
    )


def kernel(center_words, in_emb, out_emb):
    b = center_words.shape[0]
    v, d = out_emb.shape

    in_pad = jnp.pad(in_emb, ((0, 0), (0, 128 - d)))
    center_vecs = _sc_gather(in_pad, center_words)

    center_bf = center_vecs[:, :d].astype(jnp.bfloat16)
    out_bf = out_emb.astype(jnp.bfloat16)

    grid = (pl.cdiv(v, _VOCAB_TILE),)
    scores = pl.pallas_call(
        _matmul_body,
        grid=grid,
        in_specs=[
            pl.BlockSpec((b, d), lambda j: (0, 0)),
            pl.BlockSpec((_VOCAB_TILE, d), lambda j: (j, 0)),
        ],
        out_specs=pl.BlockSpec((b, _VOCAB_TILE), lambda j: (0, j)),
        out_shape=jax.ShapeDtypeStruct((b, v), jnp.float32),
        compiler_params=pltpu.CompilerParams(
            dimension_semantics=("parallel",),
        ),
    )(center_bf, out_bf)
    return scores


# speedup vs baseline: 3.3131x; 3.3052x over previous
import jax
import jax.numpy as jnp
from jax.experimental import pallas as pl
from jax.experimental.pallas import tpu as pltpu
from jax.experimental.pallas import tpu_sc as plsc

_GATHER_WINDOW = 128
_VOCAB_TILE = 1024


def _sc_gather(table, indices):
    b = indices.shape[0]
    d = table.shape[1]
    idx2d = indices.reshape(1, b)
    mesh = plsc.VectorSubcoreMesh(core_axis_name="core", subcore_axis_name="subcore")

    @pl.kernel(out_type=jax.ShapeDtypeStruct((b, d), table.dtype), mesh=mesh)
    def gather_kernel(x_hbm, i_hbm, o_hbm):
        def body(i_vmem, o_vmem):
            pltpu.sync_copy(x_hbm.at[i_vmem.at[0]], o_vmem)

        pltpu.emit_pipeline(
            body,
            grid=(b // _GATHER_WINDOW,),
            in_specs=[pl.BlockSpec((1, _GATHER_WINDOW), index_map=lambda i: (0, i))],
            out_specs=[pl.BlockSpec((_GATHER_WINDOW, d), index_map=lambda i: (i, 0))],
            core_axis_name=("core", "subcore"),
            dimension_semantics=(pltpu.PARALLEL,),
        )(i_hbm, o_hbm)

    return gather_kernel(table, idx2d)


def _matmul_body(e_ref, c_ref, o_ref):
    o_ref[...] = jax.lax.dot_general(
        e_ref[...],
        c_ref[...],
        dimension_numbers=(((1,), (1,)), ((), ())),
        preferred_element_type=jnp.float32,
        precision=jax.lax.Precision.DEFAULT,
    )


def kernel(center_words, in_emb, out_emb):
    b = center_words.shape[0]
    v, d = out_emb.shape

    in_pad = jnp.pad(in_emb, ((0, 0), (0, 128 - d)))
    center_vecs = _sc_gather(in_pad, center_words)

    center_bf = center_vecs[:, :d].astype(jnp.bfloat16)
    out_bf = out_emb.astype(jnp.bfloat16)

    num_tiles = pl.cdiv(v, _VOCAB_TILE)
    scores_t = pl.pallas_call(
        _matmul_body,
        grid=(num_tiles,),
        in_specs=[
            pl.BlockSpec((_VOCAB_TILE, d), lambda j: (j, 0)),
            pl.BlockSpec((b, d), lambda j: (0, 0)),
        ],
        out_specs=pl.BlockSpec((_VOCAB_TILE, b), lambda j: (j, 0)),
        out_shape=jax.ShapeDtypeStruct((v, b), jnp.float32),
    )(out_bf, center_bf)
    return scores_t.T
